# baseline (speedup 1.0000x reference)
import functools

import jax
import jax.numpy as jnp
from jax import lax
from jax.experimental import pallas as pl
from jax.experimental.pallas import tpu as pltpu
from jax.experimental.pallas import tpu_sc as plsc

_K = 32
_N = 10000
_M = 10000
_NW = 32
_NSEG = 10
_SR = _N // _NSEG
_RPW = 32
_SPAD = _NW * _RPW
_NC = 80
_BR = 200


def _tc_a_body(a_ref, m32_ref, cm_ref):
    v = jnp.maximum(a_ref[...], 0.0)
    rows = v.shape[0]
    cm_a = jnp.max(v[:, :9984].reshape(rows, 78, 128), axis=2)
    cm_b = jnp.max(v[:, 9984:], axis=1, keepdims=True)
    pad = jnp.full((rows, 1), -1.0, jnp.float32)
    cm = jnp.concatenate([cm_a, cm_b, pad], axis=1)
    cm_ref[...] = cm
    cmi = lax.bitcast_convert_type(cm[:, :79], jnp.int32)

    def step(i, t):
        bit = lax.shift_left(jnp.int32(1), jnp.int32(30) - i)
        cand = jnp.bitwise_or(t, bit)
        cnt = jnp.sum((cmi >= cand).astype(jnp.int32), axis=1, keepdims=True)
        return jnp.where(cnt >= _K, cand, t)

    t = lax.fori_loop(0, 31, step, jnp.zeros((rows, 1), jnp.int32))
    m32_ref[...] = lax.bitcast_convert_type(t, jnp.float32).reshape(1, 1, rows)


def _tc_a(a, seg):
    off = seg * (_SR // _BR)
    m32, cm = pl.pallas_call(
        _tc_a_body,
        grid=(_SR // _BR,),
        in_specs=[pl.BlockSpec((_BR, _M), lambda i: (i + off, 0))],
        out_specs=[
            pl.BlockSpec((1, 1, _BR), lambda i: (i, 0, 0)),
            pl.BlockSpec((_BR, _NC), lambda i: (i, 0)),
        ],
        out_shape=[
            jax.ShapeDtypeStruct((_SR // _BR, 1, _BR), jnp.float32),
            jax.ShapeDtypeStruct((_SR, _NC), jnp.float32),
        ],
    )(a)
    return m32.reshape(_SR), cm


def _sc_body(seg_base, a_hbm, m32_hbm, cm_hbm, t_hbm, row_a, row_b, cand_v,
             cid_v, m32_v, cm_v, t_v, sem_a, sem_b):
    wid = lax.axis_index("s") * 2 + lax.axis_index("c")
    base = wid * _RPW
    pltpu.sync_copy(m32_hbm.at[pl.ds(base, _RPW)], m32_v.at[pl.ds(0, _RPW)])
    pltpu.sync_copy(cm_hbm.at[pl.ds(base * _NC, _RPW * _NC)], cm_v)
    lane = lax.iota(jnp.int32, 16)
    zeros16 = jnp.zeros((16,), jnp.int32)

    def rclamp(r_local):
        return jnp.minimum(seg_base + base + r_local, _N - 1)

    def process(r_local, row_v):
        m32r = jnp.full((16,), m32_v[pl.ds(r_local, 16)][0])

        pos_c = jnp.int32(0)
        for v in range(5):
            cm16 = cm_v[pl.ds(r_local * _NC + v * 16, 16)]
            mask = jnp.logical_and(cm16 >= m32r, cm16 > 0.0)
            plsc.store_compressed(cid_v.at[pl.ds(pos_c, 16)],
                                  lane + v * 16, mask=mask)
            pos_c = pos_c + plsc.all_reduce_population_count(mask)[0]

        for i in range(0, 80, 16):
            cand_v[pl.ds(i, 16)] = zeros16

        def chunk_step(k, pos):
            j = cid_v[pl.ds(k, 16)][0]
            vals, masks = [], []
            for i in range(8):
                off = j * 128 + i * 16
                x = row_v[pl.ds(jnp.minimum(off, _M - 16), 16)]
                xr = jnp.maximum(x, 0.0)
                mask = jnp.logical_and(xr >= m32r, xr > 0.0)
                mask = jnp.logical_and(mask, lane + off < _M)
                vals.append(plsc.bitcast(xr, jnp.int32))
                masks.append(mask)
            pcs = [plsc.all_reduce_population_count(m)[0] for m in masks]
            for i in range(8):
                plsc.store_compressed(cand_v.at[pl.ds(pos, 16)],
                                      vals[i], mask=masks[i])
                pos = pos + pcs[i]
            return pos

        pos = lax.fori_loop(0, pos_c, chunk_step, jnp.int32(0))
        cand_v[pl.ds(pos, 16)] = zeros16
        n_v = (pos + 15) // 16

        def bit_step(i, t):
            cand = jnp.bitwise_or(t, lax.shift_left(jnp.int32(1),
                                                    jnp.int32(30) - i))
            pc = [plsc.all_reduce_population_count(
                cand_v[pl.ds(j * 16, 16)] >= cand)[0] for j in range(4)]
            cnt = pc[0] + pc[1] + pc[2] + pc[3]

            def cnt_step(j, c):
                x = cand_v[pl.ds(j * 16, 16)]
                return c + plsc.all_reduce_population_count(x >= cand)[0]

            cnt = lax.fori_loop(4, n_v, cnt_step, cnt)
            return jnp.where(cnt >= _K, cand, t)

        t_i = lax.fori_loop(0, 31, bit_step, jnp.int32(0))
        t_f = plsc.bitcast(jnp.full((16,), t_i, jnp.int32), jnp.float32)
        plsc.store_compressed(t_v.at[pl.ds(r_local, 16)], t_f, mask=lane == 0)

    pltpu.async_copy(a_hbm.at[rclamp(0)], row_a, sem_a)

    def pair(p, _):
        r0 = 2 * p
        pltpu.async_copy(a_hbm.at[rclamp(r0 + 1)], row_b, sem_b)
        pltpu.make_async_copy(a_hbm.at[rclamp(r0)], row_a, sem_a).wait()
        process(r0, row_a)
        pltpu.async_copy(a_hbm.at[rclamp(r0 + 2)], row_a, sem_a)
        pltpu.make_async_copy(a_hbm.at[rclamp(r0 + 1)], row_b, sem_b).wait()
        process(r0 + 1, row_b)
        return 0

    lax.fori_loop(0, _RPW // 2, pair, 0)
    pltpu.make_async_copy(a_hbm.at[rclamp(0)], row_a, sem_a).wait()
    pltpu.sync_copy(t_v.at[pl.ds(0, _RPW)], t_hbm.at[pl.ds(base, _RPW)])


def _make_sc(seg):
    @functools.partial(
        pl.kernel,
        mesh=plsc.VectorSubcoreMesh(core_axis_name="c", subcore_axis_name="s"),
        out_type=jax.ShapeDtypeStruct((_SPAD,), jnp.float32),
        compiler_params=pltpu.CompilerParams(needs_layout_passes=False),
        scratch_types=[
            pltpu.VMEM((_M,), jnp.float32),
            pltpu.VMEM((_M,), jnp.float32),
            pltpu.VMEM((_M + 32,), jnp.int32),
            pltpu.VMEM((96,), jnp.int32),
            pltpu.VMEM((_RPW + 16,), jnp.float32),
            pltpu.VMEM((_RPW * _NC,), jnp.float32),
            pltpu.VMEM((_RPW + 16,), jnp.float32),
            pltpu.SemaphoreType.DMA,
            pltpu.SemaphoreType.DMA,
        ],
    )
    def _sc_select(a_hbm, m32_hbm, cm_hbm, t_hbm, *rest):
        _sc_body(seg * _SR, a_hbm, m32_hbm, cm_hbm, t_hbm, *rest)

    return _sc_select


_SC_KERNELS = [_make_sc(s) for s in range(_NSEG)]


def _tc_b_first(a_ref, t_ref, o_ref):
    v = jnp.maximum(a_ref[...], 0.0)
    o_ref[...] = jnp.where(v >= t_ref[...], v, 0.0)


def _tc_b_chain(dummy_ref, a_ref, t_ref, o_ref):
    v = jnp.maximum(a_ref[...], 0.0)
    o_ref[...] = jnp.where(v >= t_ref[...], v, 0.0)


def _tc_b(a, t_seg, seg, prev):
    off = seg * (_SR // _BR)
    t2 = t_seg[:_SR].reshape(_SR, 1)
    if prev is None:
        return pl.pallas_call(
            _tc_b_first,
            grid=(_SR // _BR,),
            in_specs=[
                pl.BlockSpec((_BR, _M), lambda i: (i + off, 0)),
                pl.BlockSpec((_BR, 1), lambda i: (i, 0)),
            ],
            out_specs=pl.BlockSpec((_BR, _M), lambda i: (i + off, 0)),
            out_shape=jax.ShapeDtypeStruct((_N, _M), jnp.float32),
        )(a, t2)
    return pl.pallas_call(
        _tc_b_chain,
        grid=(_SR // _BR,),
        in_specs=[
            pl.BlockSpec((8, 128), lambda i: (0, 0)),
            pl.BlockSpec((_BR, _M), lambda i: (i + off, 0)),
            pl.BlockSpec((_BR, 1), lambda i: (i, 0)),
        ],
        out_specs=pl.BlockSpec((_BR, _M), lambda i: (i + off, 0)),
        out_shape=jax.ShapeDtypeStruct((_N, _M), jnp.float32),
        input_output_aliases={0: 0},
    )(prev, a, t2)


def kernel(idx, A_param):
    out = None
    for s in range(_NSEG):
        m32, cm = _tc_a(A_param, s)
        m32p = jnp.pad(m32, (0, _SPAD - _SR))
        cmp_ = jnp.pad(cm, ((0, _SPAD - _SR), (0, 0)))
        t = _SC_KERNELS[s](A_param, m32p, cmp_.reshape(_SPAD * _NC))
        out = _tc_b(A_param, t, s, out)
    return out


# speedup vs baseline: 1.0527x; 1.0527x over previous
import functools

import jax
import jax.numpy as jnp
from jax import lax
from jax.experimental import pallas as pl
from jax.experimental.pallas import tpu as pltpu
from jax.experimental.pallas import tpu_sc as plsc

_K = 32
_N = 10000
_M = 10000
_NW = 32
_NSEG = 5
_SR = _N // _NSEG
_RPW = 64
_SPAD = _NW * _RPW
_NC = 80
_BR = 200


def _tc_a_body(a_ref, m32_ref, cm_ref):
    v = jnp.maximum(a_ref[...], 0.0)
    rows = v.shape[0]
    cm_a = jnp.max(v[:, :9984].reshape(rows, 78, 128), axis=2)
    cm_b = jnp.max(v[:, 9984:], axis=1, keepdims=True)
    pad = jnp.full((rows, 1), -1.0, jnp.float32)
    cm = jnp.concatenate([cm_a, cm_b, pad], axis=1)
    cm_ref[...] = cm
    cmi = lax.bitcast_convert_type(cm[:, :79], jnp.int32)

    def step(i, t):
        bit = lax.shift_left(jnp.int32(1), jnp.int32(30) - i)
        cand = jnp.bitwise_or(t, bit)
        cnt = jnp.sum((cmi >= cand).astype(jnp.int32), axis=1, keepdims=True)
        return jnp.where(cnt >= _K, cand, t)

    t = lax.fori_loop(0, 31, step, jnp.zeros((rows, 1), jnp.int32))
    m32_ref[...] = lax.bitcast_convert_type(t, jnp.float32).reshape(1, 1, rows)


def _tc_a(a, seg):
    off = seg * (_SR // _BR)
    m32, cm = pl.pallas_call(
        _tc_a_body,
        grid=(_SR // _BR,),
        in_specs=[pl.BlockSpec((_BR, _M), lambda i: (i + off, 0))],
        out_specs=[
            pl.BlockSpec((1, 1, _BR), lambda i: (i, 0, 0)),
            pl.BlockSpec((_BR, _NC), lambda i: (i, 0)),
        ],
        out_shape=[
            jax.ShapeDtypeStruct((_SR // _BR, 1, _BR), jnp.float32),
            jax.ShapeDtypeStruct((_SR, _NC), jnp.float32),
        ],
    )(a)
    return m32.reshape(_SR), cm


def _sc_body(seg_base, a_hbm, m32_hbm, cm_hbm, t_hbm, row_a, row_b, cand_v,
             cid_v, m32_v, cm_v, t_v, sem_a, sem_b):
    wid = lax.axis_index("s") * 2 + lax.axis_index("c")
    base = wid * _RPW
    pltpu.sync_copy(m32_hbm.at[pl.ds(base, _RPW)], m32_v.at[pl.ds(0, _RPW)])
    pltpu.sync_copy(cm_hbm.at[pl.ds(base * _NC, _RPW * _NC)], cm_v)
    lane = lax.iota(jnp.int32, 16)
    zeros16 = jnp.zeros((16,), jnp.int32)

    def rclamp(r_local):
        return jnp.minimum(seg_base + base + r_local, _N - 1)

    def process(r_local, row_v):
        m32r = jnp.full((16,), m32_v[pl.ds(r_local, 16)][0])

        pos_c = jnp.int32(0)
        for v in range(5):
            cm16 = cm_v[pl.ds(r_local * _NC + v * 16, 16)]
            mask = jnp.logical_and(cm16 >= m32r, cm16 > 0.0)
            plsc.store_compressed(cid_v.at[pl.ds(pos_c, 16)],
                                  lane + v * 16, mask=mask)
            pos_c = pos_c + plsc.all_reduce_population_count(mask)[0]

        for i in range(0, 80, 16):
            cand_v[pl.ds(i, 16)] = zeros16

        def chunk_step(k, pos):
            j = cid_v[pl.ds(k, 16)][0]
            vals, masks = [], []
            for i in range(8):
                off = j * 128 + i * 16
                x = row_v[pl.ds(jnp.minimum(off, _M - 16), 16)]
                xr = jnp.maximum(x, 0.0)
                mask = jnp.logical_and(xr >= m32r, xr > 0.0)
                mask = jnp.logical_and(mask, lane + off < _M)
                vals.append(plsc.bitcast(xr, jnp.int32))
                masks.append(mask)
            pcs = [plsc.all_reduce_population_count(m)[0] for m in masks]
            for i in range(8):
                plsc.store_compressed(cand_v.at[pl.ds(pos, 16)],
                                      vals[i], mask=masks[i])
                pos = pos + pcs[i]
            return pos

        pos = lax.fori_loop(0, pos_c, chunk_step, jnp.int32(0))
        cand_v[pl.ds(pos, 16)] = zeros16
        n_v = (pos + 15) // 16

        def bit_step(i, t):
            cand = jnp.bitwise_or(t, lax.shift_left(jnp.int32(1),
                                                    jnp.int32(30) - i))
            pc = [plsc.all_reduce_population_count(
                cand_v[pl.ds(j * 16, 16)] >= cand)[0] for j in range(4)]
            cnt = pc[0] + pc[1] + pc[2] + pc[3]

            def cnt_step(j, c):
                x = cand_v[pl.ds(j * 16, 16)]
                return c + plsc.all_reduce_population_count(x >= cand)[0]

            cnt = lax.fori_loop(4, n_v, cnt_step, cnt)
            return jnp.where(cnt >= _K, cand, t)

        t_i = lax.fori_loop(0, 31, bit_step, jnp.int32(0))
        t_f = plsc.bitcast(jnp.full((16,), t_i, jnp.int32), jnp.float32)
        plsc.store_compressed(t_v.at[pl.ds(r_local, 16)], t_f, mask=lane == 0)

    pltpu.async_copy(a_hbm.at[rclamp(0)], row_a, sem_a)

    def pair(p, _):
        r0 = 2 * p
        pltpu.async_copy(a_hbm.at[rclamp(r0 + 1)], row_b, sem_b)
        pltpu.make_async_copy(a_hbm.at[rclamp(r0)], row_a, sem_a).wait()
        process(r0, row_a)
        pltpu.async_copy(a_hbm.at[rclamp(r0 + 2)], row_a, sem_a)
        pltpu.make_async_copy(a_hbm.at[rclamp(r0 + 1)], row_b, sem_b).wait()
        process(r0 + 1, row_b)
        return 0

    lax.fori_loop(0, _RPW // 2, pair, 0)
    pltpu.make_async_copy(a_hbm.at[rclamp(0)], row_a, sem_a).wait()
    pltpu.sync_copy(t_v.at[pl.ds(0, _RPW)], t_hbm.at[pl.ds(base, _RPW)])


def _make_sc(seg):
    @functools.partial(
        pl.kernel,
        mesh=plsc.VectorSubcoreMesh(core_axis_name="c", subcore_axis_name="s"),
        out_type=jax.ShapeDtypeStruct((_SPAD,), jnp.float32),
        compiler_params=pltpu.CompilerParams(needs_layout_passes=False),
        scratch_types=[
            pltpu.VMEM((_M,), jnp.float32),
            pltpu.VMEM((_M,), jnp.float32),
            pltpu.VMEM((_M + 32,), jnp.int32),
            pltpu.VMEM((96,), jnp.int32),
            pltpu.VMEM((_RPW + 16,), jnp.float32),
            pltpu.VMEM((_RPW * _NC,), jnp.float32),
            pltpu.VMEM((_RPW + 16,), jnp.float32),
            pltpu.SemaphoreType.DMA,
            pltpu.SemaphoreType.DMA,
        ],
    )
    def _sc_select(a_hbm, m32_hbm, cm_hbm, t_hbm, *rest):
        _sc_body(seg * _SR, a_hbm, m32_hbm, cm_hbm, t_hbm, *rest)

    return _sc_select


_SC_KERNELS = [_make_sc(s) for s in range(_NSEG)]


def _tc_b_first(a_ref, t_ref, o_ref):
    v = jnp.maximum(a_ref[...], 0.0)
    o_ref[...] = jnp.where(v >= t_ref[...], v, 0.0)


def _tc_b_chain(dummy_ref, a_ref, t_ref, o_ref):
    v = jnp.maximum(a_ref[...], 0.0)
    o_ref[...] = jnp.where(v >= t_ref[...], v, 0.0)


def _tc_b(a, t_seg, seg, prev):
    off = seg * (_SR // _BR)
    t2 = t_seg[:_SR].reshape(_SR, 1)
    if prev is None:
        return pl.pallas_call(
            _tc_b_first,
            grid=(_SR // _BR,),
            in_specs=[
                pl.BlockSpec((_BR, _M), lambda i: (i + off, 0)),
                pl.BlockSpec((_BR, 1), lambda i: (i, 0)),
            ],
            out_specs=pl.BlockSpec((_BR, _M), lambda i: (i + off, 0)),
            out_shape=jax.ShapeDtypeStruct((_N, _M), jnp.float32),
        )(a, t2)
    return pl.pallas_call(
        _tc_b_chain,
        grid=(_SR // _BR,),
        in_specs=[
            pl.BlockSpec((8, 128), lambda i: (0, 0)),
            pl.BlockSpec((_BR, _M), lambda i: (i + off, 0)),
            pl.BlockSpec((_BR, 1), lambda i: (i, 0)),
        ],
        out_specs=pl.BlockSpec((_BR, _M), lambda i: (i + off, 0)),
        out_shape=jax.ShapeDtypeStruct((_N, _M), jnp.float32),
        input_output_aliases={0: 0},
    )(prev, a, t2)


def kernel(idx, A_param):
    out = None
    for s in range(_NSEG):
        m32, cm = _tc_a(A_param, s)
        m32p = jnp.pad(m32, (0, _SPAD - _SR))
        cmp_ = jnp.pad(cm, ((0, _SPAD - _SR), (0, 0)))
        t = _SC_KERNELS[s](A_param, m32p, cmp_.reshape(_SPAD * _NC))
        out = _tc_b(A_param, t, s, out)
    return out
